# baseline (speedup 1.0000x reference)
import functools
import math

import jax
import jax.numpy as jnp
from jax import lax
from jax.experimental import pallas as pl
from jax.experimental.pallas import tpu as pltpu
from jax.experimental.pallas import tpu_sc as plsc

_MBLK = 512


def _dist_argmin_kernel(x_ref, cb_ref, idx_ref, esq_ref):
    k = cb_ref.shape[0]
    cb = cb_ref[...]

    @pl.when(pl.program_id(0) == 0)
    def _():
        esq_ref[...] = jnp.sum(cb * cb, axis=1)[None, :]

    x = x_ref[...]
    x_sq = jnp.sum(x * x, axis=1, keepdims=True)
    e_sq = esq_ref[...]
    dot = lax.dot_general(x, cb, (((1,), (1,)), ((), ())),
                          preferred_element_type=jnp.float32)
    dist = x_sq + e_sq - 2.0 * dot
    idx_ref[0, 0, :] = jnp.argmin(dist, axis=1).astype(jnp.int32)


def _compute_indices(x, codebook):
    n, c = x.shape
    k = codebook.shape[0]
    nblk = n // _MBLK
    idx3 = pl.pallas_call(
        _dist_argmin_kernel,
        grid=(nblk,),
        in_specs=[
            pl.BlockSpec((_MBLK, c), lambda i: (i, 0)),
            pl.BlockSpec((k, c), lambda i: (0, 0)),
        ],
        out_specs=pl.BlockSpec((1, 1, _MBLK), lambda i: (i, 0, 0)),
        out_shape=jax.ShapeDtypeStruct((nblk, 1, _MBLK), jnp.int32),
        scratch_shapes=[pltpu.VMEM((1, k), jnp.float32)],
    )(x, codebook)
    return idx3.reshape(-1)


@functools.lru_cache(maxsize=None)
def _make_sc_gather(n, k, d):
    nc, ns = 2, 16
    nw = nc * ns
    assert n % nw == 0 and d % 128 == 0
    sl = d // 128
    b_per_w = n // nw
    chunk = 96 if b_per_w % 96 == 0 else 72
    assert b_per_w % chunk == 0 and chunk % 8 == 0
    nchunk = b_per_w // chunk
    mesh = plsc.VectorSubcoreMesh(core_axis_name="c", subcore_axis_name="s")

    @functools.partial(
        pl.kernel,
        mesh=mesh,
        out_type=jax.ShapeDtypeStruct((n, sl, 128), jnp.float32),
        scratch_types=[
            pltpu.VMEM((nchunk, chunk), jnp.int32),
            pltpu.VMEM((2, chunk, sl, 128), jnp.float32),
            pltpu.SemaphoreType.DMA,
            pltpu.SemaphoreType.DMA,
            pltpu.SemaphoreType.DMA,
            pltpu.SemaphoreType.DMA,
            pltpu.SemaphoreType.DMA,
        ],
    )
    def gather_kernel(cb_hbm, idx_hbm, out_hbm, idx2d, rows_v,
                      isem, g0, g1, w0, w1):
        wid = lax.axis_index("s") * nc + lax.axis_index("c")
        base = wid * b_per_w
        gsem = (g0, g1)
        wsem = (w0, w1)
        cps = [pltpu.async_copy(idx_hbm.at[pl.ds(base + ci * chunk, chunk)],
                                idx2d.at[ci], isem)
               for ci in range(nchunk)]
        for cp in cps:
            cp.wait()
        gathers = [None] * nchunk
        pending_w = [None, None]
        gathers[0] = pltpu.async_copy(cb_hbm.at[idx2d.at[0]],
                                      rows_v.at[0], gsem[0])
        for ci in range(nchunk):
            b = ci % 2
            nb = (ci + 1) % 2
            if ci + 1 < nchunk:
                if pending_w[nb] is not None:
                    pending_w[nb].wait()
                    pending_w[nb] = None
                gathers[ci + 1] = pltpu.async_copy(
                    cb_hbm.at[idx2d.at[ci + 1]], rows_v.at[nb], gsem[nb])
            gathers[ci].wait()
            pending_w[b] = pltpu.async_copy(
                rows_v.at[b], out_hbm.at[pl.ds(base + ci * chunk, chunk)],
                wsem[b])
        for b in range(2):
            if pending_w[b] is not None:
                pending_w[b].wait()

    return gather_kernel


def kernel(in_feas, codebook):
    bq, lq, cq = in_feas.shape
    x = in_feas.reshape(-1, cq)
    n = x.shape[0]
    k, d = codebook.shape
    idx = _compute_indices(x, codebook)
    cb3 = codebook.reshape(k, d // 128, 128)
    quant = _make_sc_gather(n, k, d)(cb3, idx)
    h = int(math.sqrt(lq))
    w = lq // h
    return quant.reshape(bq, lq, cq), idx.reshape(bq, h, w)


# speedup vs baseline: 1.2938x; 1.2938x over previous
import functools
import math

import jax
import jax.numpy as jnp
from jax import lax
from jax.experimental import pallas as pl
from jax.experimental.pallas import tpu as pltpu
from jax.experimental.pallas import tpu_sc as plsc

_MBLK = 512


def _dist_argmin_kernel(x_ref, cb_ref, idx_ref, esq_ref):
    k = cb_ref.shape[0]
    cb = cb_ref[...]

    @pl.when(pl.program_id(0) == 0)
    def _():
        esq_ref[...] = jnp.sum(cb * cb, axis=1)[None, :]

    x = x_ref[...]
    x_sq = jnp.sum(x * x, axis=1, keepdims=True)
    e_sq = esq_ref[...]
    dot = lax.dot_general(x, cb, (((1,), (1,)), ((), ())),
                          preferred_element_type=jnp.float32)
    dist = x_sq + e_sq - 2.0 * dot
    idx_ref[0, 0, :] = jnp.argmin(dist, axis=1).astype(jnp.int32)


def _compute_indices(x, codebook):
    n, c = x.shape
    k = codebook.shape[0]
    nblk = n // _MBLK
    idx3 = pl.pallas_call(
        _dist_argmin_kernel,
        grid=(nblk,),
        in_specs=[
            pl.BlockSpec((_MBLK, c), lambda i: (i, 0)),
            pl.BlockSpec((k, c), lambda i: (0, 0)),
        ],
        out_specs=pl.BlockSpec((1, 1, _MBLK), lambda i: (i, 0, 0)),
        out_shape=jax.ShapeDtypeStruct((nblk, 1, _MBLK), jnp.int32),
        scratch_shapes=[pltpu.VMEM((1, k), jnp.float32)],
    )(x, codebook)
    return idx3.reshape(-1)


@functools.lru_cache(maxsize=None)
def _make_sc_gather(n, k, d):
    nc, ns = 2, 16
    nw = nc * ns
    assert n % nw == 0
    b_per_w = n // nw
    chunk = 96 if b_per_w % 96 == 0 else 72
    assert b_per_w % chunk == 0 and chunk % 8 == 0
    nchunk = b_per_w // chunk
    mesh = plsc.VectorSubcoreMesh(core_axis_name="c", subcore_axis_name="s")

    @functools.partial(
        pl.kernel,
        mesh=mesh,
        out_type=jax.ShapeDtypeStruct((n, d), jnp.float32),
        scratch_types=[
            pltpu.VMEM((nchunk, chunk), jnp.int32),
            pltpu.VMEM((2, chunk, d), jnp.float32),
            pltpu.SemaphoreType.DMA,
            pltpu.SemaphoreType.DMA,
            pltpu.SemaphoreType.DMA,
            pltpu.SemaphoreType.DMA,
            pltpu.SemaphoreType.DMA,
        ],
    )
    def gather_kernel(cb_hbm, idx_hbm, out_hbm, idx2d, rows_v,
                      isem, g0, g1, w0, w1):
        wid = lax.axis_index("s") * nc + lax.axis_index("c")
        base = wid * b_per_w
        gsem = (g0, g1)
        wsem = (w0, w1)
        cps = [pltpu.async_copy(idx_hbm.at[pl.ds(base + ci * chunk, chunk)],
                                idx2d.at[ci], isem)
               for ci in range(nchunk)]
        for cp in cps:
            cp.wait()
        gathers = [None] * nchunk
        pending_w = [None, None]
        gathers[0] = pltpu.async_copy(cb_hbm.at[idx2d.at[0]],
                                      rows_v.at[0], gsem[0])
        for ci in range(nchunk):
            b = ci % 2
            nb = (ci + 1) % 2
            if ci + 1 < nchunk:
                if pending_w[nb] is not None:
                    pending_w[nb].wait()
                    pending_w[nb] = None
                gathers[ci + 1] = pltpu.async_copy(
                    cb_hbm.at[idx2d.at[ci + 1]], rows_v.at[nb], gsem[nb])
            gathers[ci].wait()
            pending_w[b] = pltpu.async_copy(
                rows_v.at[b], out_hbm.at[pl.ds(base + ci * chunk, chunk)],
                wsem[b])
        for b in range(2):
            if pending_w[b] is not None:
                pending_w[b].wait()

    return gather_kernel


def kernel(in_feas, codebook):
    bq, lq, cq = in_feas.shape
    x = in_feas.reshape(-1, cq)
    n = x.shape[0]
    k, d = codebook.shape
    idx = _compute_indices(x, codebook)
    quant = _make_sc_gather(n, k, d)(codebook, idx)
    h = int(math.sqrt(lq))
    w = lq // h
    return quant.reshape(bq, lq, cq), idx.reshape(bq, h, w)


# speedup vs baseline: 1.3070x; 1.0102x over previous
import functools
import math

import jax
import jax.numpy as jnp
from jax import lax
from jax.experimental import pallas as pl
from jax.experimental.pallas import tpu as pltpu
from jax.experimental.pallas import tpu_sc as plsc

_MBLK = 512
_NCH = 4


def _dist_argmin_kernel(x_ref, cb_ref, idx_ref, esq_ref):
    cb = cb_ref[...]

    @pl.when(pl.program_id(0) == 0)
    def _():
        esq_ref[...] = jnp.sum(cb * cb, axis=1)[None, :]

    x = x_ref[...]
    x_sq = jnp.sum(x * x, axis=1, keepdims=True)
    e_sq = esq_ref[...]
    dot = lax.dot_general(x, cb, (((1,), (1,)), ((), ())),
                          preferred_element_type=jnp.float32)
    dist = x_sq + e_sq - 2.0 * dot
    idx_ref[0, 0, :] = jnp.argmin(dist, axis=1).astype(jnp.int32)


@functools.lru_cache(maxsize=None)
def _make_tc_scorer(n, c, k, blk0, nblk):
    return pl.pallas_call(
        _dist_argmin_kernel,
        grid=(nblk,),
        in_specs=[
            pl.BlockSpec((_MBLK, c), lambda i: (blk0 + i, 0)),
            pl.BlockSpec((k, c), lambda i: (0, 0)),
        ],
        out_specs=pl.BlockSpec((1, 1, _MBLK), lambda i: (i, 0, 0)),
        out_shape=jax.ShapeDtypeStruct((nblk, 1, _MBLK), jnp.int32),
        scratch_shapes=[pltpu.VMEM((1, k), jnp.float32)],
    )


@functools.lru_cache(maxsize=None)
def _make_sc_gather(nidx, nout, k, d):
    nc, ns = 2, 16
    nw = nc * ns
    assert nidx % nw == 0
    b_per_w = nidx // nw
    chunk = 96 if b_per_w % 96 == 0 else 72
    assert b_per_w % chunk == 0 and chunk % 8 == 0
    nchunk = b_per_w // chunk
    mesh = plsc.VectorSubcoreMesh(core_axis_name="c", subcore_axis_name="s")

    @functools.partial(
        pl.kernel,
        mesh=mesh,
        out_type=jax.ShapeDtypeStruct((nout, d), jnp.float32),
        scratch_types=[
            pltpu.VMEM((nchunk, chunk), jnp.int32),
            pltpu.VMEM((2, chunk, d), jnp.float32),
            pltpu.SemaphoreType.DMA,
            pltpu.SemaphoreType.DMA,
            pltpu.SemaphoreType.DMA,
            pltpu.SemaphoreType.DMA,
            pltpu.SemaphoreType.DMA,
        ],
    )
    def gather_kernel(cb_hbm, idx_hbm, out_hbm, idx2d, rows_v,
                      isem, g0, g1, w0, w1):
        wid = lax.axis_index("s") * nc + lax.axis_index("c")
        base = wid * b_per_w
        gsem = (g0, g1)
        wsem = (w0, w1)
        cps = [pltpu.async_copy(idx_hbm.at[pl.ds(base + ci * chunk, chunk)],
                                idx2d.at[ci], isem)
               for ci in range(nchunk)]
        for cp in cps:
            cp.wait()
        gathers = [None] * nchunk
        pending_w = [None, None]
        gathers[0] = pltpu.async_copy(cb_hbm.at[idx2d.at[0]],
                                      rows_v.at[0], gsem[0])
        for ci in range(nchunk):
            b = ci % 2
            nb = (ci + 1) % 2
            if ci + 1 < nchunk:
                if pending_w[nb] is not None:
                    pending_w[nb].wait()
                    pending_w[nb] = None
                gathers[ci + 1] = pltpu.async_copy(
                    cb_hbm.at[idx2d.at[ci + 1]], rows_v.at[nb], gsem[nb])
            gathers[ci].wait()
            pending_w[b] = pltpu.async_copy(
                rows_v.at[b], out_hbm.at[pl.ds(base + ci * chunk, chunk)],
                wsem[b])
        for b in range(2):
            if pending_w[b] is not None:
                pending_w[b].wait()

    return gather_kernel


def kernel(in_feas, codebook):
    bq, lq, cq = in_feas.shape
    x = in_feas.reshape(-1, cq)
    n = x.shape[0]
    k, d = codebook.shape
    nblk = n // _MBLK
    blk_per_ch = nblk // _NCH
    ch = blk_per_ch * _MBLK
    idxs = []
    quant = None
    for i in range(_NCH):
        scorer = _make_tc_scorer(n, cq, k, i * blk_per_ch, blk_per_ch)
        idx_i = scorer(x, codebook).reshape(-1)
        idxs.append(idx_i)
        if i == 0:
            quant = _make_sc_gather(ch, n, k, d)(codebook, idx_i)
        else:
            qi = _make_sc_gather(ch, ch, k, d)(codebook, idx_i)
            quant = lax.dynamic_update_slice(quant, qi, (i * ch, 0))
    idx = jnp.concatenate(idxs, axis=0)
    h = int(math.sqrt(lq))
    w = lq // h
    return quant.reshape(bq, lq, cq), idx.reshape(bq, h, w)
